# Initial kernel scaffold; baseline (speedup 1.0000x reference)
import jax
import jax.numpy as jnp
from jax.experimental import pallas as pl


def kernel(cross_attention, cur_disp):
    raise NotImplementedError("write your pallas kernel here")


# speedup vs baseline: 12.1608x; 12.1608x over previous
import functools

import jax
import jax.numpy as jnp
from jax import lax
from jax.experimental import pallas as pl
from jax.experimental.pallas import tpu as pltpu
from jax.experimental.pallas import tpu_sc as plsc

B, H, W = 4, 128, 256
NUM_LEVELS = 4
K = 9
CH = NUM_LEVELS * K
HALF = 128
GROUPS = HALF // 16

_mesh = plsc.VectorSubcoreMesh(core_axis_name="c", subcore_axis_name="s")


@functools.partial(
    pl.kernel,
    mesh=_mesh,
    compiler_params=pltpu.CompilerParams(use_tc_tiling_on_sc=False,
                                          needs_layout_passes=False),
    out_type=[
        jax.ShapeDtypeStruct((B, CH, H, W), jnp.float32),
        jax.ShapeDtypeStruct((B, CH, H, W), jnp.float32),
    ],
    scratch_types=[
        pltpu.VMEM((HALF, W), jnp.float32),
        pltpu.VMEM((HALF,), jnp.float32),
        pltpu.VMEM((CH, HALF), jnp.float32),
        pltpu.VMEM((CH, HALF), jnp.float32),
    ],
)
def _sc_kernel(ca_hbm, disp_hbm, out_ca_hbm, out_disp_hbm,
               rows_v, disp_v, oca_v, odisp_v):
    nc = 2
    wid = lax.axis_index("s") * nc + lax.axis_index("c")
    iota = lax.iota(jnp.int32, 16)

    def halfrow_body(h, _):
        hr = wid * 32 + h
        r = hr >> 1
        half = hr & 1
        b = r >> 7
        y = r & 127
        x0 = half * HALF

        pltpu.sync_copy(ca_hbm.at[b, y, pl.ds(x0, HALF), :], rows_v)
        pltpu.sync_copy(disp_hbm.at[b, 0, y, pl.ds(x0, HALF)], disp_v)

        def group_body(g, _):
            px = g * 16
            d = disp_v[pl.ds(px, 16)]
            x_f = (iota + (x0 + px)).astype(jnp.float32)
            pix = iota + px

            for i in range(NUM_LEVELS):
                sc = 1 << i
                w1 = W >> i
                inv = jnp.float32(1.0 / sc)
                d_i = d * inv
                xs = x_f * inv
                c = jnp.minimum(jnp.maximum(xs - d_i, 0.0),
                                jnp.float32(w1 - 1))
                fl = c.astype(jnp.int32)
                f = c - fl.astype(jnp.float32)
                one_m_f = 1.0 - f

                win = []
                for j in range(10):
                    pj = jnp.minimum(jnp.maximum(fl + (j - 4), 0), w1 - 1)
                    base = pj * sc
                    acc = plsc.load_gather(rows_v, [pix, base])
                    for m in range(1, sc):
                        acc = acc + plsc.load_gather(rows_v, [pix, base + m])
                    win.append(acc * inv)

                for k in range(-4, 5):
                    t = c + jnp.float32(k)
                    if k <= 0:
                        msk = t > 0.0
                    else:
                        rd = xs - t
                        msk = jnp.logical_and(rd > 0.0,
                                              rd < jnp.float32(w1 - 1))
                    val = one_m_f * win[k + 4] + f * win[k + 5]
                    ch = K * i + k + 4
                    oca_v[ch, pl.ds(px, 16)] = jnp.where(msk, val, 0.0)
                    odisp_v[ch, pl.ds(px, 16)] = d_i + jnp.float32(k)

        lax.fori_loop(0, GROUPS, group_body, None)

        pltpu.sync_copy(oca_v, out_ca_hbm.at[b, :, y, pl.ds(x0, HALF)])
        pltpu.sync_copy(odisp_v, out_disp_hbm.at[b, :, y, pl.ds(x0, HALF)])

    lax.fori_loop(0, 32, halfrow_body, None)


def kernel(cross_attention, cur_disp):
    return tuple(_sc_kernel(cross_attention, cur_disp))


# speedup vs baseline: 16.2921x; 1.3397x over previous
import functools

import jax
import jax.numpy as jnp
from jax import lax
from jax.experimental import pallas as pl
from jax.experimental.pallas import tpu as pltpu
from jax.experimental.pallas import tpu_sc as plsc

B, H, W = 4, 128, 256
NUM_LEVELS = 4
K = 9
CH = NUM_LEVELS * K
HALF = 128
GROUPS = HALF // 16
STEPS = 32

_mesh = plsc.VectorSubcoreMesh(core_axis_name="c", subcore_axis_name="s")


@functools.partial(
    pl.kernel,
    mesh=_mesh,
    compiler_params=pltpu.CompilerParams(use_tc_tiling_on_sc=False,
                                         needs_layout_passes=False),
    out_type=[
        jax.ShapeDtypeStruct((B, CH, H, W), jnp.float32),
        jax.ShapeDtypeStruct((B, CH, H, W), jnp.float32),
    ],
    scratch_types=[
        pltpu.VMEM((2, HALF, W), jnp.float32),
        pltpu.VMEM((2, HALF), jnp.float32),
        pltpu.VMEM((2, CH, HALF), jnp.float32),
        pltpu.VMEM((2, CH, HALF), jnp.float32),
        pltpu.SemaphoreType.DMA,
        pltpu.SemaphoreType.DMA,
        pltpu.SemaphoreType.DMA,
        pltpu.SemaphoreType.DMA,
    ],
)
def _sc_kernel(ca_hbm, disp_hbm, out_ca_hbm, out_disp_hbm,
               rows_v, disp_v, oca_v, odisp_v,
               sem_in0, sem_in1, sem_out0, sem_out1):
    nc = 2
    wid = lax.axis_index("s") * nc + lax.axis_index("c")
    iota = lax.iota(jnp.int32, 16)
    sems_in = (sem_in0, sem_in1)
    sems_out = (sem_out0, sem_out1)

    def coords(hr):
        r = hr >> 1
        half = hr & 1
        return r >> 7, r & 127, half * HALF

    def start_in(hr, buf):
        b, y, x0 = coords(hr)
        pltpu.async_copy(ca_hbm.at[b, y, pl.ds(x0, HALF), :],
                         rows_v.at[buf], sems_in[buf])
        pltpu.async_copy(disp_hbm.at[b, 0, y, pl.ds(x0, HALF)],
                         disp_v.at[buf], sems_in[buf])

    def wait_in(hr, buf):
        b, y, x0 = coords(hr)
        pltpu.make_async_copy(ca_hbm.at[b, y, pl.ds(x0, HALF), :],
                              rows_v.at[buf], sems_in[buf]).wait()
        pltpu.make_async_copy(disp_hbm.at[b, 0, y, pl.ds(x0, HALF)],
                              disp_v.at[buf], sems_in[buf]).wait()

    def start_out(hr, buf):
        b, y, x0 = coords(hr)
        pltpu.async_copy(oca_v.at[buf],
                         out_ca_hbm.at[b, :, y, pl.ds(x0, HALF)],
                         sems_out[buf])
        pltpu.async_copy(odisp_v.at[buf],
                         out_disp_hbm.at[b, :, y, pl.ds(x0, HALF)],
                         sems_out[buf])

    def wait_out(hr, buf):
        b, y, x0 = coords(hr)
        pltpu.make_async_copy(oca_v.at[buf],
                              out_ca_hbm.at[b, :, y, pl.ds(x0, HALF)],
                              sems_out[buf]).wait()
        pltpu.make_async_copy(odisp_v.at[buf],
                              out_disp_hbm.at[b, :, y, pl.ds(x0, HALF)],
                              sems_out[buf]).wait()

    def compute(buf, x0):
        rows = rows_v.at[buf]
        oca = oca_v.at[buf]
        odisp = odisp_v.at[buf]

        def group_body(g, _):
            px = g * 16
            d = disp_v.at[buf][pl.ds(px, 16)]
            x_f = (iota + (x0 + px)).astype(jnp.float32)
            pix = iota + px

            for i in range(NUM_LEVELS):
                sc = 1 << i
                w1 = W >> i
                inv = jnp.float32(1.0 / sc)
                d_i = d * inv
                xs = x_f * inv
                c = jnp.minimum(jnp.maximum(xs - d_i, 0.0),
                                jnp.float32(w1 - 1))
                fl = c.astype(jnp.int32)
                f = c - fl.astype(jnp.float32)
                one_m_f = 1.0 - f

                win = []
                for j in range(10):
                    pj = jnp.minimum(jnp.maximum(fl + (j - 4), 0), w1 - 1)
                    base = pj * sc
                    acc = plsc.load_gather(rows, [pix, base])
                    for m in range(1, sc):
                        acc = acc + plsc.load_gather(rows, [pix, base + m])
                    win.append(acc * inv)

                for k in range(-4, 5):
                    t = c + jnp.float32(k)
                    if k <= 0:
                        msk = t > 0.0
                    else:
                        rd = xs - t
                        msk = jnp.logical_and(rd > 0.0,
                                              rd < jnp.float32(w1 - 1))
                    val = one_m_f * win[k + 4] + f * win[k + 5]
                    ch = K * i + k + 4
                    oca[ch, pl.ds(px, 16)] = jnp.where(msk, val, 0.0)
                    odisp[ch, pl.ds(px, 16)] = d_i + jnp.float32(k)

        lax.fori_loop(0, GROUPS, group_body, None)

    base_hr = wid * STEPS
    start_in(base_hr, 0)

    def step_body(s2, _):
        for bufi in range(2):
            s = s2 * 2 + bufi
            hr = base_hr + s

            @pl.when(s + 1 < STEPS)
            def _():
                start_in(hr + 1, 1 - bufi)

            wait_in(hr, bufi)

            @pl.when(s >= 2)
            def _():
                wait_out(hr - 2, bufi)

            _, _, x0 = coords(hr)
            compute(bufi, x0)
            start_out(hr, bufi)

    lax.fori_loop(0, STEPS // 2, step_body, None)
    wait_out(base_hr + STEPS - 2, 0)
    wait_out(base_hr + STEPS - 1, 1)


def kernel(cross_attention, cur_disp):
    return tuple(_sc_kernel(cross_attention, cur_disp))


# speedup vs baseline: 18.2711x; 1.1215x over previous
import functools

import jax
import jax.numpy as jnp
from jax import lax
from jax.experimental import pallas as pl
from jax.experimental.pallas import tpu as pltpu
from jax.experimental.pallas import tpu_sc as plsc

B, H, W = 4, 128, 256
NUM_LEVELS = 4
K = 9
CH = NUM_LEVELS * K
HALF = 128
GROUPS = HALF // 16
STEPS = 32
CHUNK = HALF * W

_mesh = plsc.VectorSubcoreMesh(core_axis_name="c", subcore_axis_name="s")


@functools.partial(
    pl.kernel,
    mesh=_mesh,
    compiler_params=pltpu.CompilerParams(use_tc_tiling_on_sc=False,
                                         needs_layout_passes=False),
    out_type=jax.ShapeDtypeStruct((B, CH, H, W), jnp.float32),
    scratch_types=[
        pltpu.VMEM((2, CHUNK), jnp.float32),
        pltpu.VMEM((2, HALF), jnp.float32),
        pltpu.VMEM((2, CH, HALF), jnp.float32),
        pltpu.SemaphoreType.DMA,
        pltpu.SemaphoreType.DMA,
        pltpu.SemaphoreType.DMA,
        pltpu.SemaphoreType.DMA,
    ],
)
def _sc_kernel(ca_hbm, disp_hbm, out_ca_hbm,
               rows_v, disp_v, oca_v,
               sem_in0, sem_in1, sem_out0, sem_out1):
    nc = 2
    wid = lax.axis_index("s") * nc + lax.axis_index("c")
    iota = lax.iota(jnp.int32, 16)
    iota_row = iota * W
    sems_in = (sem_in0, sem_in1)
    sems_out = (sem_out0, sem_out1)

    def coords(hr):
        r = hr >> 1
        half = hr & 1
        return r >> 7, r & 127, half * HALF

    def start_in(hr, buf):
        pltpu.async_copy(ca_hbm.at[pl.ds(hr * CHUNK, CHUNK)],
                         rows_v.at[buf], sems_in[buf])
        pltpu.async_copy(disp_hbm.at[pl.ds(hr * HALF, HALF)],
                         disp_v.at[buf], sems_in[buf])

    def wait_in(hr, buf):
        pltpu.make_async_copy(ca_hbm.at[pl.ds(hr * CHUNK, CHUNK)],
                              rows_v.at[buf], sems_in[buf]).wait()
        pltpu.make_async_copy(disp_hbm.at[pl.ds(hr * HALF, HALF)],
                              disp_v.at[buf], sems_in[buf]).wait()

    def start_out(hr, buf):
        b, y, x0 = coords(hr)
        pltpu.async_copy(oca_v.at[buf],
                         out_ca_hbm.at[b, :, y, pl.ds(x0, HALF)],
                         sems_out[buf])

    def wait_out(hr, buf):
        b, y, x0 = coords(hr)
        pltpu.make_async_copy(oca_v.at[buf],
                              out_ca_hbm.at[b, :, y, pl.ds(x0, HALF)],
                              sems_out[buf]).wait()

    def compute(buf, x0):
        rows = rows_v.at[buf]
        oca = oca_v.at[buf]

        def group_body(g, _):
            px = g * 16
            d = disp_v.at[buf][pl.ds(px, 16)]
            x_f = (iota + (x0 + px)).astype(jnp.float32)
            prow = iota_row + px * W

            for i in range(NUM_LEVELS):
                sc = 1 << i
                w1 = W >> i
                inv = jnp.float32(1.0 / sc)
                d_i = d * inv
                xs = x_f * inv
                c = jnp.minimum(jnp.maximum(xs - d_i, 0.0),
                                jnp.float32(w1 - 1))
                fl = c.astype(jnp.int32)
                f = c - fl.astype(jnp.float32)
                u = xs - c

                win = []
                for j in range(10):
                    pj = jnp.minimum(jnp.maximum(fl + (j - 4), 0), w1 - 1)
                    base = prow + pj * sc
                    acc = plsc.load_gather(rows, [base])
                    for m in range(1, sc):
                        acc = acc + plsc.load_gather(rows, [base + m])
                    win.append(acc * inv)

                for k in range(-4, 5):
                    if k <= 0:
                        msk = c > jnp.float32(-k)
                    else:
                        msk = jnp.logical_and(
                            u > jnp.float32(k),
                            u < jnp.float32(w1 - 1 + k))
                    w_lo = win[k + 4]
                    val = w_lo + f * (win[k + 5] - w_lo)
                    oca[K * i + k + 4, pl.ds(px, 16)] = (
                        jnp.where(msk, val, 0.0))

        lax.fori_loop(0, GROUPS, group_body, None)

    base_hr = wid * STEPS
    start_in(base_hr, 0)

    def step_body(s2, _):
        for bufi in range(2):
            s = s2 * 2 + bufi
            hr = base_hr + s

            @pl.when(s + 1 < STEPS)
            def _():
                start_in(hr + 1, 1 - bufi)

            wait_in(hr, bufi)

            @pl.when(s >= 2)
            def _():
                wait_out(hr - 2, bufi)

            _, _, x0 = coords(hr)
            compute(bufi, x0)
            start_out(hr, bufi)

    lax.fori_loop(0, STEPS // 2, step_body, None)
    wait_out(base_hr + STEPS - 2, 0)
    wait_out(base_hr + STEPS - 1, 1)


def _disp_tc_body(disp_ref, out_ref):
    ch = pl.program_id(1)
    lvl = ch // K
    inv = 1.0 / (1 << lvl).astype(jnp.float32)
    off = (ch % K - 4).astype(jnp.float32)
    out_ref[...] = disp_ref[...] * inv + off


_disp_tc = pl.pallas_call(
    _disp_tc_body,
    grid=(B, CH),
    in_specs=[pl.BlockSpec((1, 1, H, W), lambda b, ch: (b, 0, 0, 0))],
    out_specs=pl.BlockSpec((1, 1, H, W), lambda b, ch: (b, ch, 0, 0)),
    out_shape=jax.ShapeDtypeStruct((B, CH, H, W), jnp.float32),
)


def kernel(cross_attention, cur_disp):
    ca_flat = cross_attention.reshape(-1)
    disp_flat = cur_disp.reshape(-1)
    out_ca = _sc_kernel(ca_flat, disp_flat)
    out_disp = _disp_tc(cur_disp)
    return (out_ca, out_disp)


# speedup vs baseline: 36.7075x; 2.0091x over previous
import functools

import jax
import jax.numpy as jnp
from jax import lax
from jax.experimental import pallas as pl
from jax.experimental.pallas import tpu as pltpu
from jax.experimental.pallas import tpu_sc as plsc

B, H, W = 4, 128, 256
NUM_LEVELS = 4
K = 9
CH = NUM_LEVELS * K
HALF = 128
GROUPS = HALF // 16
STEPS = 32
CHUNK = HALF * W

_mesh = plsc.VectorSubcoreMesh(core_axis_name="c", subcore_axis_name="s")


@functools.partial(
    pl.kernel,
    mesh=_mesh,
    compiler_params=pltpu.CompilerParams(use_tc_tiling_on_sc=True,
                                         needs_layout_passes=False),
    out_type=jax.ShapeDtypeStruct((B, CH, H, W), jnp.float32),
    scratch_types=[
        pltpu.VMEM((2, HALF, W), jnp.float32),
        pltpu.VMEM((2, HALF), jnp.float32),
        pltpu.VMEM((2, CH, HALF), jnp.float32),
        pltpu.SemaphoreType.DMA,
        pltpu.SemaphoreType.DMA,
        pltpu.SemaphoreType.DMA,
        pltpu.SemaphoreType.DMA,
    ],
)
def _sc_kernel(ca_hbm, disp_hbm, out_ca_hbm,
               rows_v, disp_v, oca_v,
               sem_in0, sem_in1, sem_out0, sem_out1):
    nc = 2
    wid = lax.axis_index("s") * nc + lax.axis_index("c")
    iota = lax.iota(jnp.int32, 16)
    sems_in = (sem_in0, sem_in1)
    sems_out = (sem_out0, sem_out1)

    def coords(hr):
        r = hr >> 1
        half = hr & 1
        return r >> 7, r & 127, half * HALF

    def start_in(hr, buf):
        b, y, x0 = coords(hr)
        pltpu.async_copy(ca_hbm.at[b, y, pl.ds(x0, HALF), :],
                         rows_v.at[buf], sems_in[buf])
        pltpu.async_copy(disp_hbm.at[b, 0, y, pl.ds(x0, HALF)],
                         disp_v.at[buf], sems_in[buf])

    def wait_in(hr, buf):
        b, y, x0 = coords(hr)
        pltpu.make_async_copy(ca_hbm.at[b, y, pl.ds(x0, HALF), :],
                              rows_v.at[buf], sems_in[buf]).wait()
        pltpu.make_async_copy(disp_hbm.at[b, 0, y, pl.ds(x0, HALF)],
                              disp_v.at[buf], sems_in[buf]).wait()

    def start_out(hr, buf):
        b, y, x0 = coords(hr)
        pltpu.async_copy(oca_v.at[buf],
                         out_ca_hbm.at[b, :, y, pl.ds(x0, HALF)],
                         sems_out[buf])

    def wait_out(hr, buf):
        b, y, x0 = coords(hr)
        pltpu.make_async_copy(oca_v.at[buf],
                              out_ca_hbm.at[b, :, y, pl.ds(x0, HALF)],
                              sems_out[buf]).wait()

    def compute(buf, x0):
        rows = rows_v.at[buf]
        oca = oca_v.at[buf]

        def group_body(g, _):
            px = g * 16
            d = disp_v.at[buf][pl.ds(px, 16)]
            x_f = (iota + (x0 + px)).astype(jnp.float32)
            pix = iota + px

            for i in range(NUM_LEVELS):
                sc = 1 << i
                w1 = W >> i
                inv = jnp.float32(1.0 / sc)
                d_i = d * inv
                xs = x_f * inv
                c = jnp.minimum(jnp.maximum(xs - d_i, 0.0),
                                jnp.float32(w1 - 1))
                fl = c.astype(jnp.int32)
                f = c - fl.astype(jnp.float32)
                u = xs - c

                win = []
                for j in range(10):
                    pj = jnp.minimum(jnp.maximum(fl + (j - 4), 0), w1 - 1)
                    base = pj * sc
                    acc = plsc.load_gather(rows, [pix, base])
                    for m in range(1, sc):
                        acc = acc + plsc.load_gather(rows, [pix, base + m])
                    win.append(acc * inv)

                for k in range(-4, 5):
                    if k <= 0:
                        msk = c > jnp.float32(-k)
                    else:
                        msk = jnp.logical_and(
                            u > jnp.float32(k),
                            u < jnp.float32(w1 - 1 + k))
                    w_lo = win[k + 4]
                    val = w_lo + f * (win[k + 5] - w_lo)
                    oca[K * i + k + 4, pl.ds(px, 16)] = (
                        jnp.where(msk, val, 0.0))

        lax.fori_loop(0, GROUPS, group_body, None)

    base_hr = wid * STEPS
    start_in(base_hr, 0)

    def step_body(s2, _):
        for bufi in range(2):
            s = s2 * 2 + bufi
            hr = base_hr + s

            @pl.when(s + 1 < STEPS)
            def _():
                start_in(hr + 1, 1 - bufi)

            wait_in(hr, bufi)

            @pl.when(s >= 2)
            def _():
                wait_out(hr - 2, bufi)

            _, _, x0 = coords(hr)
            compute(bufi, x0)
            start_out(hr, bufi)

    lax.fori_loop(0, STEPS // 2, step_body, None)
    wait_out(base_hr + STEPS - 2, 0)
    wait_out(base_hr + STEPS - 1, 1)


def _disp_tc_body(disp_ref, out_ref):
    ch = pl.program_id(1)
    lvl = ch // K
    inv = 1.0 / (1 << lvl).astype(jnp.float32)
    off = (ch % K - 4).astype(jnp.float32)
    out_ref[...] = disp_ref[...] * inv + off


_disp_tc = pl.pallas_call(
    _disp_tc_body,
    grid=(B, CH),
    in_specs=[pl.BlockSpec((1, 1, H, W), lambda b, ch: (b, 0, 0, 0))],
    out_specs=pl.BlockSpec((1, 1, H, W), lambda b, ch: (b, ch, 0, 0)),
    out_shape=jax.ShapeDtypeStruct((B, CH, H, W), jnp.float32),
)


def kernel(cross_attention, cur_disp):
    out_ca = _sc_kernel(cross_attention, cur_disp)
    out_disp = _disp_tc(cur_disp)
    return (out_ca, out_disp)


# speedup vs baseline: 36.7384x; 1.0008x over previous
import functools

import jax
import jax.numpy as jnp
from jax import lax
from jax.experimental import pallas as pl
from jax.experimental.pallas import tpu as pltpu
from jax.experimental.pallas import tpu_sc as plsc

B, H, W = 4, 128, 256
NUM_LEVELS = 4
K = 9
CH = NUM_LEVELS * K
HALF = 128
GROUPS = HALF // 16
STEPS = 32
CHUNK = HALF * W

_mesh = plsc.VectorSubcoreMesh(core_axis_name="c", subcore_axis_name="s")


@functools.partial(
    pl.kernel,
    mesh=_mesh,
    compiler_params=pltpu.CompilerParams(use_tc_tiling_on_sc=True,
                                         needs_layout_passes=False),
    out_type=jax.ShapeDtypeStruct((B, CH, H, W), jnp.float32),
    scratch_types=[
        pltpu.VMEM((2, HALF, W), jnp.float32),
        pltpu.VMEM((2, HALF), jnp.float32),
        pltpu.VMEM((2, CH, HALF), jnp.float32),
        pltpu.SemaphoreType.DMA,
        pltpu.SemaphoreType.DMA,
        pltpu.SemaphoreType.DMA,
        pltpu.SemaphoreType.DMA,
    ],
)
def _sc_kernel(ca_hbm, disp_hbm, out_ca_hbm,
               rows_v, disp_v, oca_v,
               sem_in0, sem_in1, sem_out0, sem_out1):
    nc = 2
    wid = lax.axis_index("s") * nc + lax.axis_index("c")
    iota = lax.iota(jnp.int32, 16)
    sems_in = (sem_in0, sem_in1)
    sems_out = (sem_out0, sem_out1)

    def coords(hr):
        r = hr >> 1
        half = hr & 1
        return r >> 7, r & 127, half * HALF

    def start_in(hr, buf):
        b, y, x0 = coords(hr)
        pltpu.async_copy(ca_hbm.at[b, y, pl.ds(x0, HALF), :],
                         rows_v.at[buf], sems_in[buf])
        pltpu.async_copy(disp_hbm.at[b, 0, y, pl.ds(x0, HALF)],
                         disp_v.at[buf], sems_in[buf])

    def wait_in(hr, buf):
        b, y, x0 = coords(hr)
        pltpu.make_async_copy(ca_hbm.at[b, y, pl.ds(x0, HALF), :],
                              rows_v.at[buf], sems_in[buf]).wait()
        pltpu.make_async_copy(disp_hbm.at[b, 0, y, pl.ds(x0, HALF)],
                              disp_v.at[buf], sems_in[buf]).wait()

    def start_out(hr, buf):
        b, y, x0 = coords(hr)
        pltpu.async_copy(oca_v.at[buf],
                         out_ca_hbm.at[b, :, y, pl.ds(x0, HALF)],
                         sems_out[buf])

    def wait_out(hr, buf):
        b, y, x0 = coords(hr)
        pltpu.make_async_copy(oca_v.at[buf],
                              out_ca_hbm.at[b, :, y, pl.ds(x0, HALF)],
                              sems_out[buf]).wait()

    def compute(buf, x0):
        rows = rows_v.at[buf]
        oca = oca_v.at[buf]

        @plsc.parallel_loop(0, GROUPS)
        def group_body(g):
            px = g * 16
            d = disp_v.at[buf][pl.ds(px, 16)]
            x_f = (iota + (x0 + px)).astype(jnp.float32)
            pix = iota + px

            for i in range(NUM_LEVELS):
                sc = 1 << i
                w1 = W >> i
                inv = jnp.float32(1.0 / sc)
                d_i = d * inv
                xs = x_f * inv
                c = jnp.minimum(jnp.maximum(xs - d_i, 0.0),
                                jnp.float32(w1 - 1))
                fl = c.astype(jnp.int32)
                f = c - fl.astype(jnp.float32)
                u = xs - c

                win = []
                for j in range(10):
                    pj = jnp.minimum(jnp.maximum(fl + (j - 4), 0), w1 - 1)
                    base = pj * sc
                    acc = plsc.load_gather(rows, [pix, base])
                    for m in range(1, sc):
                        acc = acc + plsc.load_gather(rows, [pix, base + m])
                    win.append(acc * inv)

                for k in range(-4, 5):
                    if k <= 0:
                        msk = c > jnp.float32(-k)
                    else:
                        msk = jnp.logical_and(
                            u > jnp.float32(k),
                            u < jnp.float32(w1 - 1 + k))
                    w_lo = win[k + 4]
                    val = w_lo + f * (win[k + 5] - w_lo)
                    oca[K * i + k + 4, pl.ds(px, 16)] = (
                        jnp.where(msk, val, 0.0))

    base_hr = wid * STEPS
    start_in(base_hr, 0)

    def step_body(s2, _):
        for bufi in range(2):
            s = s2 * 2 + bufi
            hr = base_hr + s

            @pl.when(s + 1 < STEPS)
            def _():
                start_in(hr + 1, 1 - bufi)

            wait_in(hr, bufi)

            @pl.when(s >= 2)
            def _():
                wait_out(hr - 2, bufi)

            _, _, x0 = coords(hr)
            compute(bufi, x0)
            start_out(hr, bufi)

    lax.fori_loop(0, STEPS // 2, step_body, None)
    wait_out(base_hr + STEPS - 2, 0)
    wait_out(base_hr + STEPS - 1, 1)


def _disp_tc_body(disp_ref, out_ref):
    ch = pl.program_id(1)
    lvl = ch // K
    inv = 1.0 / (1 << lvl).astype(jnp.float32)
    off = (ch % K - 4).astype(jnp.float32)
    out_ref[...] = disp_ref[...] * inv + off


_disp_tc = pl.pallas_call(
    _disp_tc_body,
    grid=(B, CH),
    in_specs=[pl.BlockSpec((1, 1, H, W), lambda b, ch: (b, 0, 0, 0))],
    out_specs=pl.BlockSpec((1, 1, H, W), lambda b, ch: (b, ch, 0, 0)),
    out_shape=jax.ShapeDtypeStruct((B, CH, H, W), jnp.float32),
)


def kernel(cross_attention, cur_disp):
    out_ca = _sc_kernel(cross_attention, cur_disp)
    out_disp = _disp_tc(cur_disp)
    return (out_ca, out_disp)
